# Initial kernel scaffold; baseline (speedup 1.0000x reference)
import jax
import jax.numpy as jnp
from jax.experimental import pallas as pl


def kernel(latents, num_atoms, atom_types, gt_frac_coords, lengths, angles, batch, edge_index, to_jimages, num_bonds, atom_emb, W_edge, b_edge, W1, b1, W2, b2, W3, b3):
    raise NotImplementedError("write your pallas kernel here")


# speedup vs baseline: 9.8833x; 9.8833x over previous
import numpy as np

import jax
import jax.numpy as jnp
from jax.experimental import pallas as pl
from jax.experimental.pallas import tpu as pltpu

_G = 250
_A = 40
_N = _G * _A
_EPG = 1280
_E = _G * _EPG
_HID = 128
_LAT = 128
_FCH = 256
_NRBF = 64
_NTYPES = 100
_NUM_T = 8
_NOISE_START = 0.01
_NOISE_END = 10.0
_CUTOFF = 7.0

_SCORE_NORMS = np.array([1.0, 0.9, 0.8, 0.7, 0.6, 0.5, 0.4, 0.3], dtype=np.float32)
_SIGMAS = np.exp(np.linspace(np.log(_NOISE_START), np.log(_NOISE_END), _NUM_T)).astype(np.float32)
_CENTERS = np.linspace(0.0, _CUTOFF, _NRBF).astype(np.float32)[None, :]
_RBF_DENOM = float(2.0 * (_CUTOFF / _NRBF) ** 2)


def _lattice_from_params(lengths, angles):
    ar = jnp.deg2rad(angles)
    coses = jnp.cos(ar)
    sins = jnp.sin(ar)
    val = jnp.clip((coses[:, 0] * coses[:, 1] - coses[:, 2]) / (sins[:, 0] * sins[:, 1]), -1.0, 1.0)
    gs = jnp.arccos(val)
    a, b, c = lengths[:, 0], lengths[:, 1], lengths[:, 2]
    va = jnp.stack([a * sins[:, 1], jnp.zeros_like(a), a * coses[:, 1]], axis=-1)
    vb = jnp.stack([-b * sins[:, 0] * jnp.cos(gs), b * sins[:, 0] * jnp.sin(gs), b * coses[:, 0]], axis=-1)
    vc = jnp.stack([jnp.zeros_like(c), jnp.zeros_like(c), c], axis=-1)
    return jnp.stack([va, vb, vc], axis=1)


def _graph_body(gp_ref, lat_ref, gtf_ref, nz_ref, at_ref, latz_ref, src_ref,
                dst_ref, tji_ref, aemb_ref, We_ref, be_ref, W1_ref, b1_ref,
                W2_ref, b2_ref, W3_ref, b3_ref, out_ref):
    g = pl.program_id(0)
    f32 = jnp.float32

    lat = lat_ref[0]

    def m(i, j):
        return lat[i:i + 1, j:j + 1]

    def minor(r, c):
        rs = [i for i in (0, 1, 2) if i != r]
        cs = [j for j in (0, 1, 2) if j != c]
        return m(rs[0], cs[0]) * m(rs[1], cs[1]) - m(rs[0], cs[1]) * m(rs[1], cs[0])

    det = m(0, 0) * minor(0, 0) - m(0, 1) * minor(0, 1) + m(0, 2) * minor(0, 2)
    inv_rows = []
    for i in range(3):
        row = [((-1.0) ** (i + j)) * minor(j, i) for j in range(3)]
        inv_rows.append(jnp.concatenate(row, axis=1))
    invl = jnp.concatenate(inv_rows, axis=0) / det

    gp = gp_ref[0]
    sigma = gp[:, 0:1]
    sn = gp[:, 1:2]

    gtf = gtf_ref[0]
    nz = nz_ref[0]
    gt_cart = jnp.dot(gtf, lat, preferred_element_type=f32)
    pert0 = gt_cart + sigma * nz
    frac_p = jnp.dot(pert0, invl, preferred_element_type=f32)
    frac_p = frac_p - jnp.floor(frac_p)
    pert = jnp.dot(frac_p, lat, preferred_element_type=f32)
    aligned = gtf + jnp.round(frac_p - gtf)
    aligned_c = jnp.dot(aligned, lat, preferred_element_type=f32)

    at = at_ref[0]
    type_oh = (at == jax.lax.broadcasted_iota(jnp.int32, (1, _NTYPES), 1)).astype(f32)
    emb_local = jnp.dot(type_oh, aemb_ref[...], preferred_element_type=f32)

    We = We_ref[...]
    cj = jnp.dot(emb_local, We[0:_HID, :], preferred_element_type=f32)
    ci = jnp.dot(emb_local, We[_HID:2 * _HID, :], preferred_element_type=f32)

    latv = latz_ref[0]
    lat_contrib = jnp.dot(latv, W1_ref[_HID:, :], preferred_element_type=f32) + b1_ref[...]

    src = src_ref[0]
    dst = dst_ref[0]
    iota_a = jax.lax.broadcasted_iota(jnp.int32, (1, _A), 1)
    src_oh = (src == iota_a).astype(f32)
    dst_oh = (dst == iota_a).astype(f32)

    pj = jnp.dot(src_oh, pert, preferred_element_type=f32)
    pi = jnp.dot(dst_oh, pert, preferred_element_type=f32)
    tji = tji_ref[0]
    offs = jnp.dot(tji, lat, preferred_element_type=f32)
    dvec = pi - pj + offs
    d = jnp.sqrt(jnp.sum(dvec * dvec, axis=1, keepdims=True))

    aj = jnp.dot(src_oh, aligned_c, preferred_element_type=f32)
    ai = jnp.dot(dst_oh, aligned_c, preferred_element_type=f32)
    gtv = ai - aj + offs
    gtd = jnp.sqrt(jnp.sum(gtv * gtv, axis=1, keepdims=True))

    centers = jax.lax.broadcasted_iota(jnp.int32, (1, _NRBF), 1).astype(f32) * (_CUTOFF / (_NRBF - 1))
    rbf = jnp.exp(-((d - centers) ** 2) / _RBF_DENOM)

    pre = (jnp.dot(src_oh, cj, preferred_element_type=f32)
           + jnp.dot(dst_oh, ci, preferred_element_type=f32)
           + jnp.dot(rbf, We[2 * _HID:, :], preferred_element_type=f32)
           + be_ref[...])
    ef = pre * jax.nn.sigmoid(pre)

    h1 = jnp.maximum(jnp.dot(ef, W1_ref[0:_HID, :], preferred_element_type=f32) + lat_contrib, 0.0)
    h2 = jnp.maximum(jnp.dot(h1, W2_ref[...], preferred_element_type=f32) + b2_ref[...], 0.0)
    sc = jnp.dot(h2, W3_ref[...], preferred_element_type=f32) + b3_ref[...]

    mask = (src != dst).astype(f32)
    diff = sc - (gtd - d) / sn
    le = mask * diff * diff
    gsum = jnp.sum(le, axis=(0, 1), keepdims=True)
    cnt = jnp.sum(mask, axis=(0, 1), keepdims=True)
    per_graph = gsum / jnp.maximum(cnt, 1.0)

    @pl.when(g == 0)
    def _():
        out_ref[...] = jnp.zeros_like(out_ref)

    out_ref[...] += per_graph * (1.0 / _G)


def kernel(latents, num_atoms, atom_types, gt_frac_coords, lengths, angles,
           batch, edge_index, to_jimages, num_bonds, atom_emb, W_edge, b_edge,
           W1, b1, W2, b2, W3, b3):
    f32 = jnp.float32
    time_steps = jax.random.randint(jax.random.key(42), (_G,), 0, _NUM_T)
    noise = jax.random.normal(jax.random.key(43), (_N, 3), dtype=f32)
    sigma_g = jnp.asarray(_SIGMAS)[time_steps]
    sn_g = jnp.asarray(_SCORE_NORMS)[time_steps]
    lattice = _lattice_from_params(lengths, angles)

    gp = jnp.stack([sigma_g, sn_g] + [jnp.zeros((_G,), f32)] * 6, axis=-1).reshape(_G, 1, 8)

    base = (jnp.arange(_G, dtype=jnp.int32) * _A)[:, None]
    src_l = (edge_index[0].reshape(_G, _EPG) - base).reshape(_G, _EPG, 1)
    dst_l = (edge_index[1].reshape(_G, _EPG) - base).reshape(_G, _EPG, 1)
    tji = to_jimages.astype(f32).reshape(_G, _EPG, 3)

    gtf = gt_frac_coords.reshape(_G, _A, 3)
    nz = noise.reshape(_G, _A, 3)
    at = atom_types.astype(jnp.int32).reshape(_G, _A, 1)
    latz = latents.reshape(_G, 1, _LAT)

    def whole(shape):
        return pl.BlockSpec(shape, lambda g: (0,) * len(shape))

    out = pl.pallas_call(
        _graph_body,
        grid=(_G,),
        in_specs=[
            pl.BlockSpec((1, 1, 8), lambda g: (g, 0, 0)),
            pl.BlockSpec((1, 3, 3), lambda g: (g, 0, 0)),
            pl.BlockSpec((1, _A, 3), lambda g: (g, 0, 0)),
            pl.BlockSpec((1, _A, 3), lambda g: (g, 0, 0)),
            pl.BlockSpec((1, _A, 1), lambda g: (g, 0, 0)),
            pl.BlockSpec((1, 1, _LAT), lambda g: (g, 0, 0)),
            pl.BlockSpec((1, _EPG, 1), lambda g: (g, 0, 0)),
            pl.BlockSpec((1, _EPG, 1), lambda g: (g, 0, 0)),
            pl.BlockSpec((1, _EPG, 3), lambda g: (g, 0, 0)),
            whole((_NTYPES, _HID)),
            whole((2 * _HID + _NRBF, _HID)),
            whole((1, _HID)),
            whole((_LAT + _HID, _FCH)),
            whole((1, _FCH)),
            whole((_FCH, _FCH)),
            whole((1, _FCH)),
            whole((_FCH, 1)),
            whole((1, 1)),
        ],
        out_specs=pl.BlockSpec((1, 1), lambda g: (0, 0)),
        out_shape=jax.ShapeDtypeStruct((1, 1), f32),
        compiler_params=pltpu.CompilerParams(
            dimension_semantics=("arbitrary",),
        ),
    )(gp, lattice, gtf, nz, at, latz, src_l, dst_l, tji, atom_emb, W_edge,
      b_edge.reshape(1, _HID), W1, b1.reshape(1, _FCH), W2,
      b2.reshape(1, _FCH), W3, b3.reshape(1, 1))
    return out[0, 0]
